# baseline (speedup 1.0000x reference)
import functools

import jax
import jax.numpy as jnp
from jax import lax
from jax.experimental import pallas as pl
from jax.experimental.pallas import tpu as pltpu
from jax.experimental.pallas import tpu_sc as plsc

_B = 16384
_D = 64
_NT = 5
_CHUNK = 128
_NBUF = 6


@functools.cache
def _build():
    info = plsc.get_sparse_core_info()
    nc, ns = info.num_cores, info.num_subcores
    nw = nc * ns
    b_per_w = _B // nw
    n_chunks = b_per_w // _CHUNK
    n_tasks = _NT * n_chunks
    mesh = plsc.VectorSubcoreMesh(core_axis_name="c", subcore_axis_name="s")

    @functools.partial(
        pl.kernel,
        mesh=mesh,
        out_type=jax.ShapeDtypeStruct((_NT, _B, 2 * _D), jnp.float32),
        scratch_types=(
            [pltpu.VMEM((n_tasks, _CHUNK), jnp.int32)]
            + [pltpu.VMEM((_CHUNK, 2 * _D), jnp.float32) for _ in range(_NBUF)]
            + [pltpu.SemaphoreType.DMA for _ in range(2 * _NBUF)]
        ),
    )
    def node_embedding(idx_h, w_cat, w_sub, w_elem, w_brand, w_item, out_h,
                       idx_v, *bufs_and_sems):
        rows = bufs_and_sems[:_NBUF]
        gsems = bufs_and_sems[_NBUF:2 * _NBUF]
        wsems = bufs_and_sems[2 * _NBUF:]
        tabs = [w_cat, w_sub, w_elem, w_brand, w_item]
        wid = lax.axis_index("s") * nc + lax.axis_index("c")
        base = wid * b_per_w

        pltpu.sync_copy(idx_h.at[wid], idx_v)

        inflight = [None] * _NBUF
        writes = [None] * _NBUF

        def start(j):
            b = j % _NBUF
            if writes[b] is not None:
                writes[b].wait()
                writes[b] = None
            t = j // n_chunks
            inflight[b] = pltpu.async_copy(
                tabs[t].at[idx_v.at[j]], rows[b], gsems[b])

        lookahead = _NBUF // 2
        for i in range(min(lookahead, n_tasks)):
            start(i)
        for i in range(n_tasks):
            t, c = i // n_chunks, i % n_chunks
            b = i % _NBUF
            inflight[b].wait()
            writes[b] = pltpu.async_copy(
                rows[b],
                out_h.at[t, pl.ds(base + c * _CHUNK, _CHUNK), :],
                wsems[b])
            j = i + lookahead
            if j < n_tasks:
                start(j)
        for b in range(_NBUF):
            if writes[b] is not None:
                writes[b].wait()

    return node_embedding, nw, n_chunks


def kernel(categories, sub_categories, elements, brands, product_id_remapped,
           W_cat, W_sub, W_elem, W_brand, W_item):
    fn, nw, n_chunks = _build()
    idx = jnp.stack([categories, sub_categories, elements, brands,
                     product_id_remapped]).astype(jnp.int32)
    pair = idx >> 1
    pair = pair.reshape(_NT, nw, n_chunks, _CHUNK).transpose(1, 0, 2, 3)
    pair = pair.reshape(nw, _NT * n_chunks, _CHUNK)
    wide = fn(pair,
              W_cat.reshape(-1, 2 * _D), W_sub.reshape(-1, 2 * _D),
              W_elem.reshape(-1, 2 * _D), W_brand.reshape(-1, 2 * _D),
              W_item.reshape(-1, 2 * _D))
    odd = (idx & 1).astype(bool)[:, :, None]
    half = jnp.where(odd, wide[:, :, _D:], wide[:, :, :_D])
    return half.transpose(1, 0, 2).reshape(_B, _NT * _D)


# speedup vs baseline: 1.0855x; 1.0855x over previous
import functools

import jax
import jax.numpy as jnp
from jax import lax
from jax.experimental import pallas as pl
from jax.experimental.pallas import tpu as pltpu
from jax.experimental.pallas import tpu_sc as plsc

_B = 16384
_D = 64
_NT = 5
_CHUNK = 128
_NBUF = 6


@functools.cache
def _build():
    info = plsc.get_sparse_core_info()
    nc, ns = info.num_cores, info.num_subcores
    nw = nc * ns
    b_per_w = _B // nw
    n_chunks = b_per_w // _CHUNK
    n_tasks = _NT * n_chunks
    mesh = plsc.VectorSubcoreMesh(core_axis_name="c", subcore_axis_name="s")

    @functools.partial(
        pl.kernel,
        mesh=mesh,
        out_type=jax.ShapeDtypeStruct((_NT, _B, 2 * _D), jnp.float32),
        scratch_types=(
            [pltpu.VMEM((n_tasks, _CHUNK), jnp.int32)]
            + [pltpu.VMEM((_CHUNK, 2 * _D), jnp.float32) for _ in range(_NBUF)]
            + [pltpu.SemaphoreType.DMA for _ in range(2 * _NBUF)]
        ),
    )
    def node_embedding(idx_h, w_cat, w_sub, w_elem, w_brand, w_item, out_h,
                       idx_v, *bufs_and_sems):
        rows = bufs_and_sems[:_NBUF]
        gsems = bufs_and_sems[_NBUF:2 * _NBUF]
        wsems = bufs_and_sems[2 * _NBUF:]
        tabs = [w_cat, w_sub, w_elem, w_brand, w_item]
        wid = lax.axis_index("s") * nc + lax.axis_index("c")
        base = wid * b_per_w

        pltpu.sync_copy(idx_h.at[wid], idx_v)

        inflight = [None] * _NBUF
        writes = [None] * _NBUF

        def start(j):
            b = j % _NBUF
            if writes[b] is not None:
                writes[b].wait()
                writes[b] = None
            t = j // n_chunks
            inflight[b] = pltpu.async_copy(
                tabs[t].at[idx_v.at[j]], rows[b], gsems[b])

        lookahead = _NBUF // 2
        for i in range(min(lookahead, n_tasks)):
            start(i)
        for i in range(n_tasks):
            t, c = i // n_chunks, i % n_chunks
            b = i % _NBUF
            inflight[b].wait()
            writes[b] = pltpu.async_copy(
                rows[b],
                out_h.at[t, pl.ds(base + c * _CHUNK, _CHUNK), :],
                wsems[b])
            j = i + lookahead
            if j < n_tasks:
                start(j)
        for b in range(_NBUF):
            if writes[b] is not None:
                writes[b].wait()

    return node_embedding, nw, n_chunks


def kernel(categories, sub_categories, elements, brands, product_id_remapped,
           W_cat, W_sub, W_elem, W_brand, W_item):
    fn, nw, n_chunks = _build()
    idx = jnp.stack([categories, sub_categories, elements, brands,
                     product_id_remapped]).astype(jnp.int32)
    idx = idx.reshape(_NT, nw, n_chunks, _CHUNK).transpose(1, 0, 2, 3)
    idx = idx.reshape(nw, _NT * n_chunks, _CHUNK)
    padw = ((0, 0), (0, _D))
    wide = fn(idx,
              jnp.pad(W_cat, padw), jnp.pad(W_sub, padw),
              jnp.pad(W_elem, padw), jnp.pad(W_brand, padw),
              jnp.pad(W_item, padw))
    half = wide[:, :, :_D]
    return half.transpose(1, 0, 2).reshape(_B, _NT * _D)


# speedup vs baseline: 1.5965x; 1.4708x over previous
import functools

import jax
import jax.numpy as jnp
from jax import lax
from jax.experimental import pallas as pl
from jax.experimental.pallas import tpu as pltpu
from jax.experimental.pallas import tpu_sc as plsc

_B = 16384
_D = 64
_NT = 5
_G = 16


@functools.cache
def _build():
    info = plsc.get_sparse_core_info()
    nc, ns = info.num_cores, info.num_subcores
    nw = nc * ns
    b_per_w = _B // nw
    n_groups = b_per_w // _G
    mesh = plsc.VectorSubcoreMesh(core_axis_name="c", subcore_axis_name="s")

    @functools.partial(
        pl.kernel,
        mesh=mesh,
        out_type=jax.ShapeDtypeStruct((_NT, nw, 2, b_per_w // 2, _D),
                                      jnp.float32),
        compiler_params=pltpu.CompilerParams(use_tc_tiling_on_sc=True,
                                             needs_layout_passes=False),
        scratch_types=(
            [pltpu.VMEM((b_per_w,), jnp.int32)]
            + [pltpu.VMEM((b_per_w // 2, _D), jnp.float32) for _ in range(2)]
            + [pltpu.VMEM((_G, 8, _D), jnp.float32) for _ in range(2)]
            + [pltpu.SemaphoreType.DMA for _ in range(2)]
            + [pltpu.SemaphoreType.DMA]
            + [pltpu.SemaphoreType.DMA for _ in range(2)]
        ),
    )
    def node_embedding(idx_h, w_cat, w_sub, w_elem, w_brand, w_item, out_h,
                       idx_v, stage0, stage1, blk0, blk1, *sems):
        stages = [stage0, stage1]
        blks = [blk0, blk1]
        gsems = sems[0:2]
        isem = sems[2]
        osems = sems[3:5]
        tabs = [w_cat, w_sub, w_elem, w_brand, w_item]
        wid = lax.axis_index("s") * nc + lax.axis_index("c")

        owrite = [None, None]

        def gather_group(tab, q, base, g):
            vec = idx_v[pl.ds(base + g * _G, _G)]
            gv = vec >> 3
            for u in range(_G):
                pltpu.async_copy(tab.at[gv[u]], blks[q].at[u], gsems[q])

        def gdrain(q):
            pltpu.make_async_copy(tabs[0].at[pl.ds(0, _G)], blks[q],
                                  gsems[q]).wait()

        def extract(stage, q, base, g):
            vec = idx_v[pl.ds(base + g * _G, _G)]
            for u in range(_G):
                r = vec[u] & 7
                for k in range(_D // 16):
                    stage[g * _G + u, pl.ds(16 * k, 16)] = (
                        blks[q][u, r, pl.ds(16 * k, 16)])

        hgroups = n_groups // 2
        for t in range(_NT):
            tab = tabs[t]
            pltpu.async_copy(
                idx_h.at[wid, pl.ds(t * b_per_w, b_per_w)], idx_v,
                isem).wait()
            for h in range(2):
                p = (2 * t + h) % 2
                base = h * (b_per_w // 2)
                gather_group(tab, 0, base, 0)
                if owrite[p] is not None:
                    owrite[p].wait()
                    owrite[p] = None
                stage = stages[p]

                def pair(j, _, tab=tab, stage=stage, base=base):
                    g = j * 2
                    gather_group(tab, 1, base, g + 1)
                    gdrain(0)
                    extract(stage, 0, base, g)

                    @pl.when(g + 2 < hgroups)
                    def _():
                        gather_group(tab, 0, base, g + 2)
                    gdrain(1)
                    extract(stage, 1, base, g + 1)
                    return ()

                lax.fori_loop(0, hgroups // 2, pair, ())

                owrite[p] = pltpu.async_copy(
                    stage, out_h.at[t, wid, h], osems[p])
        for p in range(2):
            if owrite[p] is not None:
                owrite[p].wait()

    return node_embedding, nw, b_per_w


def kernel(categories, sub_categories, elements, brands, product_id_remapped,
           W_cat, W_sub, W_elem, W_brand, W_item):
    fn, nw, b_per_w = _build()
    idx = jnp.stack([categories, sub_categories, elements, brands,
                     product_id_remapped]).astype(jnp.int32)
    idx = idx.reshape(_NT, nw, b_per_w).transpose(1, 0, 2)
    idx = idx.reshape(nw, _NT * b_per_w)
    parts = fn(idx,
               W_cat.reshape(-1, 8, _D), W_sub.reshape(-1, 8, _D),
               W_elem.reshape(-1, 8, _D), W_brand.reshape(-1, 8, _D),
               W_item.reshape(-1, 8, _D))
    return parts.reshape(_NT, _B, _D).transpose(1, 0, 2).reshape(
        _B, _NT * _D)


# speedup vs baseline: 1.6392x; 1.0268x over previous
import functools

import jax
import jax.numpy as jnp
from jax import lax
from jax.experimental import pallas as pl
from jax.experimental.pallas import tpu as pltpu
from jax.experimental.pallas import tpu_sc as plsc

_B = 16384
_D = 64
_NT = 5
_G = 16
_Q = 4
_H = 256


@functools.cache
def _build():
    info = plsc.get_sparse_core_info()
    nc, ns = info.num_cores, info.num_subcores
    nw = nc * ns
    b_per_w = _B // nw
    n_h = b_per_w // _H
    hgroups = _H // _G
    mesh = plsc.VectorSubcoreMesh(core_axis_name="c", subcore_axis_name="s")

    @functools.partial(
        pl.kernel,
        mesh=mesh,
        out_type=jax.ShapeDtypeStruct((_NT, nw, n_h, _H, _D), jnp.float32),
        compiler_params=pltpu.CompilerParams(use_tc_tiling_on_sc=True,
                                             needs_layout_passes=False),
        scratch_types=(
            [pltpu.VMEM((_H,), jnp.int32)]
            + [pltpu.VMEM((_H, _D), jnp.float32)]
            + [pltpu.VMEM((_G, 8, _D), jnp.float32) for _ in range(_Q)]
            + [pltpu.SemaphoreType.DMA for _ in range(_Q)]
            + [pltpu.SemaphoreType.DMA]
            + [pltpu.SemaphoreType.DMA]
        ),
    )
    def node_embedding(idx_h, w_cat, w_sub, w_elem, w_brand, w_item, out_h,
                       idx_v, stage, *rest):
        blks = rest[:_Q]
        gsems = rest[_Q:2 * _Q]
        isem = rest[2 * _Q]
        osem = rest[2 * _Q + 1]
        tabs = [w_cat, w_sub, w_elem, w_brand, w_item]
        wid = lax.axis_index("s") * nc + lax.axis_index("c")

        def gather_group(tab, q, g):
            vec = idx_v[pl.ds(g * _G, _G)]
            gv = vec >> 3
            for u in range(_G):
                pltpu.async_copy(tab.at[gv[u]], blks[q].at[u], gsems[q])

        def gdrain(q):
            pltpu.make_async_copy(tabs[0].at[pl.ds(0, _G)], blks[q],
                                  gsems[q]).wait()

        def extract(q, g):
            vec = idx_v[pl.ds(g * _G, _G)]
            for u in range(_G):
                r = vec[u] & 7
                for k in range(_D // 16):
                    stage[g * _G + u, pl.ds(16 * k, 16)] = (
                        blks[q][u, r, pl.ds(16 * k, 16)])

        for t in range(_NT):
            tab = tabs[t]

            def hbody(h, _, tab=tab, t=t):
                pltpu.async_copy(
                    idx_h.at[wid, pl.ds(t * b_per_w + h * _H, _H)], idx_v,
                    isem).wait()
                for m in range(_Q - 1):
                    gather_group(tab, m, m)

                def quad(j, _, tab=tab):
                    g0 = j * _Q
                    for m in range(_Q):
                        g = g0 + m

                        @pl.when(g + _Q - 1 < hgroups)
                        def _():
                            gather_group(tab, (m + _Q - 1) % _Q,
                                         g + _Q - 1)
                        gdrain(m)
                        extract(m, g)
                    return ()

                lax.fori_loop(0, hgroups // _Q, quad, ())
                pltpu.async_copy(
                    stage, out_h.at[t, wid, h], osem).wait()
                return ()

            lax.fori_loop(0, n_h, hbody, ())

    return node_embedding, nw, b_per_w


def kernel(categories, sub_categories, elements, brands, product_id_remapped,
           W_cat, W_sub, W_elem, W_brand, W_item):
    fn, nw, b_per_w = _build()
    idx = jnp.stack([categories, sub_categories, elements, brands,
                     product_id_remapped]).astype(jnp.int32)
    idx = idx.reshape(_NT, nw, b_per_w).transpose(1, 0, 2)
    idx = idx.reshape(nw, _NT * b_per_w)
    parts = fn(idx,
               W_cat.reshape(-1, 8, _D), W_sub.reshape(-1, 8, _D),
               W_elem.reshape(-1, 8, _D), W_brand.reshape(-1, 8, _D),
               W_item.reshape(-1, 8, _D))
    return parts.reshape(_NT, _B, _D).transpose(1, 0, 2).reshape(
        _B, _NT * _D)
